# Initial kernel scaffold; baseline (speedup 1.0000x reference)
import jax
import jax.numpy as jnp
from jax.experimental import pallas as pl


def kernel(input_data, prediction):
    raise NotImplementedError("write your pallas kernel here")


# speedup vs baseline: 79.9477x; 79.9477x over previous
import jax
import jax.numpy as jnp
from jax.experimental import pallas as pl

N_RAW = 5000
N_PAD = 5120
N_CLS = 80
J_BLK = 256
R_BLK = 256

INPUT_SIZE = 512.0
SCORE_THRESHOLD = 0.3
IOU_THRESHOLD = 0.45

_ORG_H, _ORG_W = 640, 960
_RATIO = min(INPUT_SIZE / _ORG_W, INPUT_SIZE / _ORG_H)
_DW = (INPUT_SIZE - _RATIO * _ORG_W) / 2.0
_DH = (INPUT_SIZE - _RATIO * _ORG_H) / 2.0

_f32 = jnp.float32


def _iota(shape, dim):
    return jax.lax.broadcasted_iota(jnp.int32, shape, dim).astype(_f32)


def _postprocess_kernel(pred_ref, meta_ref):
    predT = pred_ref[:, :]
    x = predT[0:1, :]
    y = predT[1:2, :]
    w = predT[2:3, :]
    h = predT[3:4, :]
    conf = predT[4:5, :]
    prob = predT[5:5 + N_CLS, :]

    xmin = x - w * 0.5
    ymin = y - h * 0.5
    xmax = x + w * 0.5
    ymax = y + h * 0.5
    ratio = _f32(_RATIO)
    x1 = (xmin - _f32(_DW)) / ratio
    x2 = (xmax - _f32(_DW)) / ratio
    y1 = (ymin - _f32(_DH)) / ratio
    y2 = (ymax - _f32(_DH)) / ratio
    x1c = jnp.maximum(x1, _f32(0.0))
    y1c = jnp.maximum(y1, _f32(0.0))
    x2c = jnp.minimum(x2, _f32(_ORG_W - 1.0))
    y2c = jnp.minimum(y2, _f32(_ORG_H - 1.0))
    invalid = (x1c > x2c) | (y1c > y2c)
    x1c = jnp.where(invalid, _f32(0.0), x1c)
    y1c = jnp.where(invalid, _f32(0.0), y1c)
    x2c = jnp.where(invalid, _f32(0.0), x2c)
    y2c = jnp.where(invalid, _f32(0.0), y2c)
    area = (x2c - x1c) * (y2c - y1c)
    bscale = jnp.sqrt(jnp.maximum(area, _f32(0.0)))
    scale_mask = bscale > _f32(0.0)

    pmax = jnp.max(prob, axis=0, keepdims=True)
    iota80c = _iota((N_CLS, 1), 0)
    cls = jnp.min(jnp.where(prob == pmax, iota80c, _f32(N_CLS)), axis=0,
                  keepdims=True)
    score0 = conf * pmax
    mask = scale_mask & (score0 > _f32(SCORE_THRESHOLD))
    s = jnp.where(mask, score0, _f32(0.0))
    pos_f = jnp.where(s > _f32(0.0), _f32(1.0), _f32(0.0))

    meta_ref[0:1, :] = x1c
    meta_ref[1:2, :] = y1c
    meta_ref[2:3, :] = x2c
    meta_ref[3:4, :] = y2c
    meta_ref[4:5, :] = area
    meta_ref[5:6, :] = s
    meta_ref[6:7, :] = cls
    meta_ref[7:8, :] = pos_f


def _ranks_kernel(meta_ref, chunk_ref, rk_ref):
    pid = pl.program_id(0)
    s = meta_ref[5:6, :]
    cls = meta_ref[6:7, :]
    s_chunk = chunk_ref[5:6, :]
    c_chunk = chunk_ref[6:7, :]

    eye = jnp.where(
        jax.lax.broadcasted_iota(jnp.int32, (J_BLK, J_BLK), 0)
        == jax.lax.broadcasted_iota(jnp.int32, (J_BLK, J_BLK), 1),
        _f32(1.0), _f32(0.0))
    s_col = jax.lax.dot_general(eye, s_chunk, (((1,), (1,)), ((), ())),
                                preferred_element_type=_f32,
                                precision=jax.lax.Precision.HIGHEST)
    c_col = jax.lax.dot_general(eye, c_chunk, (((1,), (1,)), ((), ())),
                                preferred_element_type=_f32,
                                precision=jax.lax.Precision.HIGHEST)
    j0_f = pid.astype(_f32) * _f32(J_BLK)
    i_col = _iota((J_BLK, 1), 0) + j0_f
    idx_row = _iota((1, N_PAD), 1)

    before = (s_col > s) | ((s_col == s) & (i_col < idx_row))
    d_rank = jnp.sum(jnp.where(before, _f32(1.0), _f32(0.0)),
                     axis=0, keepdims=True)
    same = before & (c_col == cls)
    d_crk = jnp.sum(jnp.where(same, _f32(1.0), _f32(0.0)),
                    axis=0, keepdims=True)
    part = jnp.concatenate([d_rank, d_crk], axis=0)

    @pl.when(pid == 0)
    def _():
        rk_ref[:, :] = jnp.zeros((2, N_PAD), _f32)

    rk_ref[:, :] += part


def _nms_loop_kernel(meta_ref, rk_ref, y6k_ref):
    x1c = meta_ref[0:1, :]
    y1c = meta_ref[1:2, :]
    x2c = meta_ref[2:3, :]
    y2c = meta_ref[3:4, :]
    area = meta_ref[4:5, :]
    s = meta_ref[5:6, :]
    cls = meta_ref[6:7, :]
    pos_f = meta_ref[7:8, :]
    crk = rk_ref[1:2, :]

    pos = pos_f > _f32(0.5)
    iota80c = _iota((N_CLS, 1), 0)
    onehot_c = jnp.where(cls == iota80c, _f32(1.0), _f32(0.0))
    max_crk = jnp.max(jnp.where(pos, crk, _f32(-1.0)))
    n_steps = (max_crk + _f32(1.0)).astype(jnp.int32)
    geomT = jnp.concatenate([x1c, y1c, x2c, y2c, area], axis=0)

    def body(r, keep):
        r_f = r.astype(_f32)
        rankeq = jnp.where((crk == r_f) & pos, _f32(1.0), _f32(0.0))
        mask_rc = rankeq * onehot_c
        data6 = jnp.concatenate([geomT, keep], axis=0)
        active = jax.lax.dot_general(mask_rc, data6, (((1,), (1,)), ((), ())),
                                     preferred_element_type=_f32,
                                precision=jax.lax.Precision.HIGHEST)
        partner = jax.lax.dot_general(active, onehot_c,
                                      (((0,), (0,)), ((), ())),
                                      preferred_element_type=_f32,
                                precision=jax.lax.Precision.HIGHEST)
        lux = jnp.maximum(partner[0:1, :], x1c)
        luy = jnp.maximum(partner[1:2, :], y1c)
        rdx = jnp.minimum(partner[2:3, :], x2c)
        rdy = jnp.minimum(partner[3:4, :], y2c)
        iw = jnp.maximum(rdx - lux, _f32(0.0))
        ih = jnp.maximum(rdy - luy, _f32(0.0))
        inter = iw * ih
        union = partner[4:5, :] + area - inter
        iou = inter / (union + _f32(1e-9))
        sup = ((iou > _f32(IOU_THRESHOLD)) & (partner[5:6, :] > _f32(0.5))
               & (crk > r_f))
        return keep * jnp.where(sup, _f32(0.0), _f32(1.0))

    keep = jax.lax.fori_loop(0, n_steps, body, pos_f)

    y6k_ref[0:1, :] = x1c * keep
    y6k_ref[1:2, :] = y1c * keep
    y6k_ref[2:3, :] = x2c * keep
    y6k_ref[3:4, :] = y2c * keep
    y6k_ref[4:5, :] = s * keep
    y6k_ref[5:6, :] = cls * keep
    y6k_ref[6:8, :] = jnp.zeros((2, N_PAD), _f32)


def _permute_kernel(rk_ref, y6k_ref, out_ref):
    pid = pl.program_id(0)
    rank = rk_ref[0:1, :]
    y6k = y6k_ref[:, :]
    r_col = _iota((R_BLK, 1), 0) + pid.astype(_f32) * _f32(R_BLK)
    p_blk = jnp.where(rank == r_col, _f32(1.0), _f32(0.0))
    out_ref[:, :] = jax.lax.dot_general(p_blk, y6k, (((1,), (1,)), ((), ())),
                                        preferred_element_type=_f32,
                                precision=jax.lax.Precision.HIGHEST)


@jax.jit
def kernel(input_data, prediction):
    del input_data
    pred_pad = jnp.zeros((96, N_PAD), jnp.float32)
    pred_pad = pred_pad.at[:85, :N_RAW].set(prediction.T)

    meta = pl.pallas_call(
        _postprocess_kernel,
        out_shape=jax.ShapeDtypeStruct((8, N_PAD), _f32),
    )(pred_pad)

    n_j = N_PAD // J_BLK
    rk = pl.pallas_call(
        _ranks_kernel,
        grid=(n_j,),
        in_specs=[
            pl.BlockSpec((8, N_PAD), lambda j: (0, 0)),
            pl.BlockSpec((8, J_BLK), lambda j: (0, j)),
        ],
        out_specs=pl.BlockSpec((2, N_PAD), lambda j: (0, 0)),
        out_shape=jax.ShapeDtypeStruct((2, N_PAD), _f32),
    )(meta, meta)

    y6k = pl.pallas_call(
        _nms_loop_kernel,
        out_shape=jax.ShapeDtypeStruct((8, N_PAD), _f32),
    )(meta, rk)

    n_r = N_PAD // R_BLK
    out = pl.pallas_call(
        _permute_kernel,
        grid=(n_r,),
        in_specs=[
            pl.BlockSpec((2, N_PAD), lambda r: (0, 0)),
            pl.BlockSpec((8, N_PAD), lambda r: (0, 0)),
        ],
        out_specs=pl.BlockSpec((R_BLK, 8), lambda r: (r, 0)),
        out_shape=jax.ShapeDtypeStruct((N_PAD, 8), _f32),
    )(rk, y6k)
    return out[:N_RAW, :6]


# speedup vs baseline: 139.1766x; 1.7408x over previous
import jax
import jax.numpy as jnp
from jax.experimental import pallas as pl
from jax.experimental.pallas import tpu as pltpu

N_RAW = 5000
N_PAD = 5120
N_CLS = 80
J_BLK = 256
R_BLK = 256
K_WIN = 128

INPUT_SIZE = 512.0
SCORE_THRESHOLD = 0.3
IOU_THRESHOLD = 0.45

_ORG_H, _ORG_W = 640, 960
_RATIO = min(INPUT_SIZE / _ORG_W, INPUT_SIZE / _ORG_H)
_DW = (INPUT_SIZE - _RATIO * _ORG_W) / 2.0
_DH = (INPUT_SIZE - _RATIO * _ORG_H) / 2.0

_f32 = jnp.float32


def _iota(shape, dim):
    return jax.lax.broadcasted_iota(jnp.int32, shape, dim).astype(_f32)


def _postprocess_kernel(pred_ref, meta_ref):
    predT = pred_ref[:, :]
    x = predT[0:1, :]
    y = predT[1:2, :]
    w = predT[2:3, :]
    h = predT[3:4, :]
    conf = predT[4:5, :]
    prob = predT[5:5 + N_CLS, :]

    xmin = x - w * 0.5
    ymin = y - h * 0.5
    xmax = x + w * 0.5
    ymax = y + h * 0.5
    ratio = _f32(_RATIO)
    x1 = (xmin - _f32(_DW)) / ratio
    x2 = (xmax - _f32(_DW)) / ratio
    y1 = (ymin - _f32(_DH)) / ratio
    y2 = (ymax - _f32(_DH)) / ratio
    x1c = jnp.maximum(x1, _f32(0.0))
    y1c = jnp.maximum(y1, _f32(0.0))
    x2c = jnp.minimum(x2, _f32(_ORG_W - 1.0))
    y2c = jnp.minimum(y2, _f32(_ORG_H - 1.0))
    invalid = (x1c > x2c) | (y1c > y2c)
    x1c = jnp.where(invalid, _f32(0.0), x1c)
    y1c = jnp.where(invalid, _f32(0.0), y1c)
    x2c = jnp.where(invalid, _f32(0.0), x2c)
    y2c = jnp.where(invalid, _f32(0.0), y2c)
    area = (x2c - x1c) * (y2c - y1c)
    bscale = jnp.sqrt(jnp.maximum(area, _f32(0.0)))
    scale_mask = bscale > _f32(0.0)

    pmax = jnp.max(prob, axis=0, keepdims=True)
    iota80c = _iota((N_CLS, 1), 0)
    cls = jnp.min(jnp.where(prob == pmax, iota80c, _f32(N_CLS)), axis=0,
                  keepdims=True)
    score0 = conf * pmax
    mask = scale_mask & (score0 > _f32(SCORE_THRESHOLD))
    s = jnp.where(mask, score0, _f32(0.0))
    pos_f = jnp.where(s > _f32(0.0), _f32(1.0), _f32(0.0))

    meta_ref[0:1, :] = x1c
    meta_ref[1:2, :] = y1c
    meta_ref[2:3, :] = x2c
    meta_ref[3:4, :] = y2c
    meta_ref[4:5, :] = area
    meta_ref[5:6, :] = s
    meta_ref[6:7, :] = cls
    meta_ref[7:8, :] = pos_f


def _ranks_kernel(meta_ref, chunk_ref, rk_ref):
    pid = pl.program_id(0)
    s = meta_ref[5:6, :]
    cls = meta_ref[6:7, :]
    s_chunk = chunk_ref[5:6, :]
    c_chunk = chunk_ref[6:7, :]

    eye = jnp.where(
        jax.lax.broadcasted_iota(jnp.int32, (J_BLK, J_BLK), 0)
        == jax.lax.broadcasted_iota(jnp.int32, (J_BLK, J_BLK), 1),
        _f32(1.0), _f32(0.0))
    s_col = jax.lax.dot_general(eye, s_chunk, (((1,), (1,)), ((), ())),
                                preferred_element_type=_f32,
                                precision=jax.lax.Precision.HIGHEST)
    c_col = jax.lax.dot_general(eye, c_chunk, (((1,), (1,)), ((), ())),
                                preferred_element_type=_f32,
                                precision=jax.lax.Precision.HIGHEST)
    j0_f = pid.astype(_f32) * _f32(J_BLK)
    i_col = _iota((J_BLK, 1), 0) + j0_f
    idx_row = _iota((1, N_PAD), 1)

    before = (s_col > s) | ((s_col == s) & (i_col < idx_row))
    d_rank = jnp.sum(jnp.where(before, _f32(1.0), _f32(0.0)),
                     axis=0, keepdims=True)
    same = before & (c_col == cls)
    d_crk = jnp.sum(jnp.where(same, _f32(1.0), _f32(0.0)),
                    axis=0, keepdims=True)
    part = jnp.concatenate([d_rank, d_crk], axis=0)

    @pl.when(pid == 0)
    def _():
        rk_ref[:, :] = jnp.zeros((2, N_PAD), _f32)

    rk_ref[:, :] += part


def _nms_loop_kernel(meta_ref, rk_ref, y6k_ref,
                     bx1_ref, by1_ref, bx2_ref, by2_ref, kb_ref):
    x1c = meta_ref[0:1, :]
    y1c = meta_ref[1:2, :]
    x2c = meta_ref[2:3, :]
    y2c = meta_ref[3:4, :]
    s = meta_ref[5:6, :]
    cls = meta_ref[6:7, :]
    pos_f = meta_ref[7:8, :]
    crk = rk_ref[1:2, :]

    pos = pos_f > _f32(0.5)
    iota80c = _iota((N_CLS, 1), 0)
    onehot_c = jnp.where(cls == iota80c, _f32(1.0), _f32(0.0))
    max_crk = jnp.max(jnp.where(pos, crk, _f32(-1.0)))
    n_steps = (max_crk + _f32(1.0)).astype(jnp.int32)
    n_win = (n_steps + K_WIN - 1) // K_WIN
    rw_col = _iota((K_WIN, 1), 0)
    c_lanes = (((1,), (1,)), ((), ()))
    c_cls = (((1,), (0,)), ((), ()))
    hi = jax.lax.Precision.HIGHEST

    def window(wb, keep):
        r0_f = wb.astype(_f32) * _f32(K_WIN)
        maskR = jnp.where(crk == (rw_col + r0_f), _f32(1.0), _f32(0.0))

        def bucket(chan):
            return jax.lax.dot_general(maskR * chan, onehot_c, c_lanes,
                                       preferred_element_type=_f32,
                                       precision=hi)

        bx1 = bucket(x1c)
        by1 = bucket(y1c)
        bx2 = bucket(x2c)
        by2 = bucket(y2c)
        kb0 = bucket(keep)
        bx1_ref[:, :] = bx1
        by1_ref[:, :] = by1
        bx2_ref[:, :] = bx2
        by2_ref[:, :] = by2
        kb_ref[:, :] = kb0
        area_b = (bx2 - bx1) * (by2 - by1)
        n_in = jnp.minimum(n_steps - wb * K_WIN, K_WIN)

        def inner(rw, carry):
            ax1 = bx1_ref[pl.ds(rw, 1), :]
            ay1 = by1_ref[pl.ds(rw, 1), :]
            ax2 = bx2_ref[pl.ds(rw, 1), :]
            ay2 = by2_ref[pl.ds(rw, 1), :]
            ak = kb_ref[pl.ds(rw, 1), :]
            kb = kb_ref[:, :]
            lux = jnp.maximum(ax1, bx1)
            luy = jnp.maximum(ay1, by1)
            rdx = jnp.minimum(ax2, bx2)
            rdy = jnp.minimum(ay2, by2)
            iw = jnp.maximum(rdx - lux, _f32(0.0))
            ih = jnp.maximum(rdy - luy, _f32(0.0))
            inter = iw * ih
            a_area = (ax2 - ax1) * (ay2 - ay1)
            union = a_area + area_b - inter
            iou = inter / (union + _f32(1e-9))
            after = rw_col > rw.astype(_f32)
            sup = (iou > _f32(IOU_THRESHOLD)) & (ak > _f32(0.5)) & after
            kb_ref[:, :] = kb * jnp.where(sup, _f32(0.0), _f32(1.0))
            return carry

        jax.lax.fori_loop(0, n_in, inner, 0)
        kbf = kb_ref[:, :]
        xk = jax.lax.dot_general(kbf, onehot_c, c_cls,
                                 preferred_element_type=_f32,
                                 precision=hi)
        in_w = jnp.sum(maskR, axis=0, keepdims=True)
        keep_w = jnp.sum(maskR * xk, axis=0, keepdims=True)
        keep1 = jnp.where(in_w > _f32(0.5), keep_w, keep)

        def cross(k):
            px1 = jax.lax.dot_general(bx1, onehot_c, c_cls,
                                      preferred_element_type=_f32,
                                      precision=hi)
            py1 = jax.lax.dot_general(by1, onehot_c, c_cls,
                                      preferred_element_type=_f32,
                                      precision=hi)
            px2 = jax.lax.dot_general(bx2, onehot_c, c_cls,
                                      preferred_element_type=_f32,
                                      precision=hi)
            py2 = jax.lax.dot_general(by2, onehot_c, c_cls,
                                      preferred_element_type=_f32,
                                      precision=hi)
            parea = (px2 - px1) * (py2 - py1)
            lux = jnp.maximum(px1, x1c)
            luy = jnp.maximum(py1, y1c)
            rdx = jnp.minimum(px2, x2c)
            rdy = jnp.minimum(py2, y2c)
            iw = jnp.maximum(rdx - lux, _f32(0.0))
            ih = jnp.maximum(rdy - luy, _f32(0.0))
            inter = iw * ih
            area_i = (x2c - x1c) * (y2c - y1c)
            union = parea + area_i - inter
            iou = inter / (union + _f32(1e-9))
            supc = ((iou > _f32(IOU_THRESHOLD)) & (xk > _f32(0.5))
                    & (crk > (rw_col + r0_f)))
            any_sup = jnp.max(jnp.where(supc, _f32(1.0), _f32(0.0)),
                              axis=0, keepdims=True)
            return k * (_f32(1.0) - any_sup)

        return jax.lax.cond(wb + 1 < n_win, cross, lambda k: k, keep1)

    keep = jax.lax.fori_loop(0, n_win, window, pos_f)

    y6k_ref[0:1, :] = x1c * keep
    y6k_ref[1:2, :] = y1c * keep
    y6k_ref[2:3, :] = x2c * keep
    y6k_ref[3:4, :] = y2c * keep
    y6k_ref[4:5, :] = s * keep
    y6k_ref[5:6, :] = cls * keep
    y6k_ref[6:8, :] = jnp.zeros((2, N_PAD), _f32)


def _permute_kernel(rk_ref, y6k_ref, out_ref):
    pid = pl.program_id(0)
    rank = rk_ref[0:1, :]
    y6k = y6k_ref[:, :]
    r_col = _iota((R_BLK, 1), 0) + pid.astype(_f32) * _f32(R_BLK)
    p_blk = jnp.where(rank == r_col, _f32(1.0), _f32(0.0))
    out_ref[:, :] = jax.lax.dot_general(p_blk, y6k, (((1,), (1,)), ((), ())),
                                        preferred_element_type=_f32,
                                precision=jax.lax.Precision.HIGHEST)


@jax.jit
def kernel(input_data, prediction):
    del input_data
    pred_pad = jnp.zeros((96, N_PAD), jnp.float32)
    pred_pad = pred_pad.at[:85, :N_RAW].set(prediction.T)

    meta = pl.pallas_call(
        _postprocess_kernel,
        out_shape=jax.ShapeDtypeStruct((8, N_PAD), _f32),
    )(pred_pad)

    n_j = N_PAD // J_BLK
    rk = pl.pallas_call(
        _ranks_kernel,
        grid=(n_j,),
        in_specs=[
            pl.BlockSpec((8, N_PAD), lambda j: (0, 0)),
            pl.BlockSpec((8, J_BLK), lambda j: (0, j)),
        ],
        out_specs=pl.BlockSpec((2, N_PAD), lambda j: (0, 0)),
        out_shape=jax.ShapeDtypeStruct((2, N_PAD), _f32),
    )(meta, meta)

    y6k = pl.pallas_call(
        _nms_loop_kernel,
        out_shape=jax.ShapeDtypeStruct((8, N_PAD), _f32),
        scratch_shapes=[pltpu.VMEM((K_WIN, N_CLS), _f32)
                        for _ in range(5)],
    )(meta, rk)

    n_r = N_PAD // R_BLK
    out = pl.pallas_call(
        _permute_kernel,
        grid=(n_r,),
        in_specs=[
            pl.BlockSpec((2, N_PAD), lambda r: (0, 0)),
            pl.BlockSpec((8, N_PAD), lambda r: (0, 0)),
        ],
        out_specs=pl.BlockSpec((R_BLK, 8), lambda r: (r, 0)),
        out_shape=jax.ShapeDtypeStruct((N_PAD, 8), _f32),
    )(rk, y6k)
    return out[:N_RAW, :6]


# speedup vs baseline: 215.8547x; 1.5509x over previous
import jax
import jax.numpy as jnp
from jax.experimental import pallas as pl
from jax.experimental.pallas import tpu as pltpu
from jax.experimental.pallas import tpu_sc as plsc

N_RAW = 5000
N_PAD = 5120
N_CLS = 80
J_BLK = 256
R_BLK = 256
K_WIN = 128

INPUT_SIZE = 512.0
SCORE_THRESHOLD = 0.3
IOU_THRESHOLD = 0.45

_ORG_H, _ORG_W = 640, 960
_RATIO = min(INPUT_SIZE / _ORG_W, INPUT_SIZE / _ORG_H)
_DW = (INPUT_SIZE - _RATIO * _ORG_W) / 2.0
_DH = (INPUT_SIZE - _RATIO * _ORG_H) / 2.0

_f32 = jnp.float32


def _iota(shape, dim):
    return jax.lax.broadcasted_iota(jnp.int32, shape, dim).astype(_f32)


def _postprocess_kernel(pred_ref, meta_ref):
    predT = pred_ref[:, :]
    x = predT[0:1, :]
    y = predT[1:2, :]
    w = predT[2:3, :]
    h = predT[3:4, :]
    conf = predT[4:5, :]
    prob = predT[5:5 + N_CLS, :]

    xmin = x - w * 0.5
    ymin = y - h * 0.5
    xmax = x + w * 0.5
    ymax = y + h * 0.5
    ratio = _f32(_RATIO)
    x1 = (xmin - _f32(_DW)) / ratio
    x2 = (xmax - _f32(_DW)) / ratio
    y1 = (ymin - _f32(_DH)) / ratio
    y2 = (ymax - _f32(_DH)) / ratio
    x1c = jnp.maximum(x1, _f32(0.0))
    y1c = jnp.maximum(y1, _f32(0.0))
    x2c = jnp.minimum(x2, _f32(_ORG_W - 1.0))
    y2c = jnp.minimum(y2, _f32(_ORG_H - 1.0))
    invalid = (x1c > x2c) | (y1c > y2c)
    x1c = jnp.where(invalid, _f32(0.0), x1c)
    y1c = jnp.where(invalid, _f32(0.0), y1c)
    x2c = jnp.where(invalid, _f32(0.0), x2c)
    y2c = jnp.where(invalid, _f32(0.0), y2c)
    area = (x2c - x1c) * (y2c - y1c)
    bscale = jnp.sqrt(jnp.maximum(area, _f32(0.0)))
    scale_mask = bscale > _f32(0.0)

    pmax = jnp.max(prob, axis=0, keepdims=True)
    iota80c = _iota((N_CLS, 1), 0)
    cls = jnp.min(jnp.where(prob == pmax, iota80c, _f32(N_CLS)), axis=0,
                  keepdims=True)
    score0 = conf * pmax
    mask = scale_mask & (score0 > _f32(SCORE_THRESHOLD))
    s = jnp.where(mask, score0, _f32(0.0))
    pos_f = jnp.where(s > _f32(0.0), _f32(1.0), _f32(0.0))

    meta_ref[0:1, :] = x1c
    meta_ref[1:2, :] = y1c
    meta_ref[2:3, :] = x2c
    meta_ref[3:4, :] = y2c
    meta_ref[4:5, :] = area
    meta_ref[5:6, :] = s
    meta_ref[6:7, :] = cls
    meta_ref[7:8, :] = pos_f


def _ranks_kernel(meta_ref, chunk_ref, rk_ref):
    pid = pl.program_id(0)
    s = meta_ref[5:6, :]
    cls = meta_ref[6:7, :]
    s_chunk = chunk_ref[5:6, :]
    c_chunk = chunk_ref[6:7, :]

    eye = jnp.where(
        jax.lax.broadcasted_iota(jnp.int32, (J_BLK, J_BLK), 0)
        == jax.lax.broadcasted_iota(jnp.int32, (J_BLK, J_BLK), 1),
        _f32(1.0), _f32(0.0))
    s_col = jax.lax.dot_general(eye, s_chunk, (((1,), (1,)), ((), ())),
                                preferred_element_type=_f32,
                                precision=jax.lax.Precision.HIGHEST)
    c_col = jax.lax.dot_general(eye, c_chunk, (((1,), (1,)), ((), ())),
                                preferred_element_type=_f32,
                                precision=jax.lax.Precision.HIGHEST)
    j0_f = pid.astype(_f32) * _f32(J_BLK)
    i_col = _iota((J_BLK, 1), 0) + j0_f
    idx_row = _iota((1, N_PAD), 1)

    before = (s_col > s) | ((s_col == s) & (i_col < idx_row))
    d_rank = jnp.sum(jnp.where(before, _f32(1.0), _f32(0.0)),
                     axis=0, keepdims=True)
    same = before & (c_col == cls)
    d_crk = jnp.sum(jnp.where(same, _f32(1.0), _f32(0.0)),
                    axis=0, keepdims=True)
    part = jnp.concatenate([d_rank, d_crk], axis=0)

    @pl.when(pid == 0)
    def _():
        rk_ref[:, :] = jnp.zeros((2, N_PAD), _f32)

    rk_ref[:, :] += part


def _nms_loop_kernel(meta_ref, rk_ref, y6k_ref,
                     bx1_ref, by1_ref, bx2_ref, by2_ref, kb_ref):
    x1c = meta_ref[0:1, :]
    y1c = meta_ref[1:2, :]
    x2c = meta_ref[2:3, :]
    y2c = meta_ref[3:4, :]
    s = meta_ref[5:6, :]
    cls = meta_ref[6:7, :]
    pos_f = meta_ref[7:8, :]
    crk = rk_ref[1:2, :]

    pos = pos_f > _f32(0.5)
    iota80c = _iota((N_CLS, 1), 0)
    onehot_c = jnp.where(cls == iota80c, _f32(1.0), _f32(0.0))
    max_crk = jnp.max(jnp.where(pos, crk, _f32(-1.0)))
    n_steps = (max_crk + _f32(1.0)).astype(jnp.int32)
    n_win = (n_steps + K_WIN - 1) // K_WIN
    rw_col = _iota((K_WIN, 1), 0)
    c_lanes = (((1,), (1,)), ((), ()))
    c_cls = (((1,), (0,)), ((), ()))
    hi = jax.lax.Precision.HIGHEST

    def window(wb, keep):
        r0_f = wb.astype(_f32) * _f32(K_WIN)
        maskR = jnp.where(crk == (rw_col + r0_f), _f32(1.0), _f32(0.0))

        def bucket(chan):
            return jax.lax.dot_general(maskR * chan, onehot_c, c_lanes,
                                       preferred_element_type=_f32,
                                       precision=hi)

        bx1 = bucket(x1c)
        by1 = bucket(y1c)
        bx2 = bucket(x2c)
        by2 = bucket(y2c)
        kb0 = bucket(keep)
        bx1_ref[:, :] = bx1
        by1_ref[:, :] = by1
        bx2_ref[:, :] = bx2
        by2_ref[:, :] = by2
        kb_ref[:, :] = kb0
        area_b = (bx2 - bx1) * (by2 - by1)
        n_in = jnp.minimum(n_steps - wb * K_WIN, K_WIN)

        def inner(rw, carry):
            ax1 = bx1_ref[pl.ds(rw, 1), :]
            ay1 = by1_ref[pl.ds(rw, 1), :]
            ax2 = bx2_ref[pl.ds(rw, 1), :]
            ay2 = by2_ref[pl.ds(rw, 1), :]
            ak = kb_ref[pl.ds(rw, 1), :]
            kb = kb_ref[:, :]
            lux = jnp.maximum(ax1, bx1)
            luy = jnp.maximum(ay1, by1)
            rdx = jnp.minimum(ax2, bx2)
            rdy = jnp.minimum(ay2, by2)
            iw = jnp.maximum(rdx - lux, _f32(0.0))
            ih = jnp.maximum(rdy - luy, _f32(0.0))
            inter = iw * ih
            a_area = (ax2 - ax1) * (ay2 - ay1)
            union = a_area + area_b - inter
            iou = inter / (union + _f32(1e-9))
            after = rw_col > rw.astype(_f32)
            sup = (iou > _f32(IOU_THRESHOLD)) & (ak > _f32(0.5)) & after
            kb_ref[:, :] = kb * jnp.where(sup, _f32(0.0), _f32(1.0))
            return carry

        jax.lax.fori_loop(0, n_in, inner, 0)
        kbf = kb_ref[:, :]
        xk = jax.lax.dot_general(kbf, onehot_c, c_cls,
                                 preferred_element_type=_f32)
        in_w = jnp.sum(maskR, axis=0, keepdims=True)
        keep_w = jnp.sum(maskR * xk, axis=0, keepdims=True)
        keep1 = jnp.where(in_w > _f32(0.5), keep_w, keep)

        def cross(k):
            px1 = jax.lax.dot_general(bx1, onehot_c, c_cls,
                                      preferred_element_type=_f32,
                                      precision=hi)
            py1 = jax.lax.dot_general(by1, onehot_c, c_cls,
                                      preferred_element_type=_f32,
                                      precision=hi)
            px2 = jax.lax.dot_general(bx2, onehot_c, c_cls,
                                      preferred_element_type=_f32,
                                      precision=hi)
            py2 = jax.lax.dot_general(by2, onehot_c, c_cls,
                                      preferred_element_type=_f32,
                                      precision=hi)
            parea = (px2 - px1) * (py2 - py1)
            lux = jnp.maximum(px1, x1c)
            luy = jnp.maximum(py1, y1c)
            rdx = jnp.minimum(px2, x2c)
            rdy = jnp.minimum(py2, y2c)
            iw = jnp.maximum(rdx - lux, _f32(0.0))
            ih = jnp.maximum(rdy - luy, _f32(0.0))
            inter = iw * ih
            area_i = (x2c - x1c) * (y2c - y1c)
            union = parea + area_i - inter
            iou = inter / (union + _f32(1e-9))
            supc = ((iou > _f32(IOU_THRESHOLD)) & (xk > _f32(0.5))
                    & (crk > (rw_col + r0_f)))
            any_sup = jnp.max(jnp.where(supc, _f32(1.0), _f32(0.0)),
                              axis=0, keepdims=True)
            return k * (_f32(1.0) - any_sup)

        return jax.lax.cond(wb + 1 < n_win, cross, lambda k: k, keep1)

    keep = jax.lax.fori_loop(0, n_win, window, pos_f)

    y6k_ref[0:1, :] = x1c * keep
    y6k_ref[1:2, :] = y1c * keep
    y6k_ref[2:3, :] = x2c * keep
    y6k_ref[3:4, :] = y2c * keep
    y6k_ref[4:5, :] = s * keep
    y6k_ref[5:6, :] = cls * keep
    y6k_ref[6:8, :] = jnp.zeros((2, N_PAD), _f32)


def _permute_kernel(rk_ref, y6k_ref, out_ref):
    pid = pl.program_id(0)
    rank = rk_ref[0:1, :]
    y6k = y6k_ref[:, :]
    r_col = _iota((R_BLK, 1), 0) + pid.astype(_f32) * _f32(R_BLK)
    p_blk = jnp.where(rank == r_col, _f32(1.0), _f32(0.0))
    out_ref[:, :] = jax.lax.dot_general(p_blk, y6k, (((1,), (1,)), ((), ())),
                                        preferred_element_type=_f32)


_SC_NW = 32
_SC_ROWS = N_PAD // _SC_NW


def _sc_scatter_kernel(rows_hbm, rank_hbm, out_hbm, idx_v, rows_v, sem):
    wid = (jax.lax.axis_index("s") * 2 + jax.lax.axis_index("c")).astype(
        jnp.int32)
    base = wid * _SC_ROWS
    pltpu.sync_copy(rank_hbm.at[pl.ds(base, _SC_ROWS)], idx_v)
    pltpu.sync_copy(rows_hbm.at[pl.ds(base, _SC_ROWS)], rows_v)
    pltpu.async_copy(rows_v, out_hbm.at[idx_v], sem).wait()


@jax.jit
def kernel(input_data, prediction):
    del input_data
    pred_pad = jnp.zeros((96, N_PAD), jnp.float32)
    pred_pad = pred_pad.at[:85, :N_RAW].set(prediction.T)

    meta = pl.pallas_call(
        _postprocess_kernel,
        out_shape=jax.ShapeDtypeStruct((8, N_PAD), _f32),
    )(pred_pad)

    n_j = N_PAD // J_BLK
    rk = pl.pallas_call(
        _ranks_kernel,
        grid=(n_j,),
        in_specs=[
            pl.BlockSpec((8, N_PAD), lambda j: (0, 0)),
            pl.BlockSpec((8, J_BLK), lambda j: (0, j)),
        ],
        out_specs=pl.BlockSpec((2, N_PAD), lambda j: (0, 0)),
        out_shape=jax.ShapeDtypeStruct((2, N_PAD), _f32),
    )(meta, meta)

    y6k = pl.pallas_call(
        _nms_loop_kernel,
        out_shape=jax.ShapeDtypeStruct((8, N_PAD), _f32),
        scratch_shapes=[pltpu.VMEM((K_WIN, N_CLS), _f32)
                        for _ in range(5)],
    )(meta, rk)

    rows = jnp.zeros((N_PAD, 128), _f32).at[:, :8].set(y6k.T)
    rank_i32 = rk[0].astype(jnp.int32)
    mesh = plsc.VectorSubcoreMesh(core_axis_name="c", subcore_axis_name="s")
    out = pl.kernel(
        _sc_scatter_kernel,
        mesh=mesh,
        out_type=jax.ShapeDtypeStruct((N_PAD, 128), _f32),
        scratch_types=[
            pltpu.VMEM((_SC_ROWS,), jnp.int32),
            pltpu.VMEM((_SC_ROWS, 128), _f32),
            pltpu.SemaphoreType.DMA,
        ],
    )(rows, rank_i32)
    return out[:N_RAW, :6]
